# baseline (speedup 1.0000x reference)
import jax
import jax.numpy as jnp
from jax import lax
from jax.experimental import pallas as pl
from jax.experimental.pallas import tpu as pltpu
from jax.experimental.pallas import tpu_sc as plsc

_VOCAB = 1000
_EMBED = 64
_BATCH = 16384

_NC = 2
_NS = 16
_NW = _NC * _NS
_BPW = _BATCH // _NW
_CHUNK = 128
_NCHUNK = _BPW // _CHUNK
_L = 16


def _gather_body(idx0_hbm, idx1_hbm, table_hbm, out_hbm,
                 i0_v, i1_v, flat_v, rows_v, sem):
    wid = lax.axis_index("s") * _NC + lax.axis_index("c")
    base = wid * _BPW
    pltpu.sync_copy(idx0_hbm.at[pl.ds(base, _BPW)], i0_v)
    pltpu.sync_copy(idx1_hbm.at[pl.ds(base, _BPW)], i1_v)
    for j in range(_NCHUNK):
        for k in range(_CHUNK // _L):
            s = pl.ds(j * _CHUNK + k * _L, _L)
            flat_v[j, pl.ds(k * _L, _L)] = i0_v[s] * _VOCAB + i1_v[s]
    copies = [
        pltpu.async_copy(table_hbm.at[flat_v.at[j]],
                         rows_v.at[pl.ds(j * _CHUNK, _CHUNK)], sem)
        for j in range(_NCHUNK)
    ]
    for c in copies:
        c.wait()
    pltpu.sync_copy(rows_v, out_hbm.at[pl.ds(base, _BPW)])


@jax.jit
def kernel(idx0, idx1, lookup):
    table = lookup.reshape(_VOCAB * _VOCAB, _EMBED)
    run = pl.kernel(
        _gather_body,
        out_type=jax.ShapeDtypeStruct((_BATCH, _EMBED), jnp.float32),
        mesh=plsc.VectorSubcoreMesh(core_axis_name="c", subcore_axis_name="s"),
        compiler_params=pltpu.CompilerParams(use_tc_tiling_on_sc=False),
        scratch_types=[
            pltpu.VMEM((_BPW,), jnp.int32),
            pltpu.VMEM((_BPW,), jnp.int32),
            pltpu.VMEM((_NCHUNK, _CHUNK), jnp.int32),
            pltpu.VMEM((_BPW, _EMBED), jnp.float32),
            pltpu.SemaphoreType.DMA,
        ],
    )
    return run(idx0, idx1, table)


# speedup vs baseline: 2.7199x; 2.7199x over previous
import jax
import jax.numpy as jnp
from jax import lax
from jax.experimental import pallas as pl
from jax.experimental.pallas import tpu as pltpu
from jax.experimental.pallas import tpu_sc as plsc

_VOCAB = 1000
_EMBED = 64
_BATCH = 16384

_NC = 2
_NS = 16
_NW = _NC * _NS
_BPW = _BATCH // _NW
_RING = 4
_L = 16


def _gather_body(idx0_hbm, idx1_hbm, table_hbm, out_hbm,
                 i0_v, i1_v, piece_v, rows_v, sem):
    wid = lax.axis_index("s") * _NC + lax.axis_index("c")
    base = wid * _BPW
    pltpu.sync_copy(idx0_hbm.at[pl.ds(base, _BPW)], i0_v.at[pl.ds(0, _BPW)])
    pltpu.sync_copy(idx1_hbm.at[pl.ds(base, _BPW)], i1_v.at[pl.ds(0, _BPW)])

    lane = lax.iota(jnp.int32, _L)

    def fire(b):
        i0 = i0_v[pl.ds(b, _L)][0]
        i1 = i1_v[pl.ds(b, _L)][0]
        ct = pl.multiple_of(lax.shift_right_logical(i1, 7) * 128, 128)
        pltpu.make_async_copy(
            table_hbm.at[i0, :, pl.ds(ct, 128)],
            piece_v.at[b & (_RING - 1)], sem,
        ).start()

    def wait_slot(b):
        pltpu.make_async_copy(
            table_hbm.at[0, :, pl.ds(pl.multiple_of(b * 0, 128), 128)],
            piece_v.at[b & (_RING - 1)], sem,
        ).wait()

    def extract(b):
        slot = (b & (_RING - 1)) + lane * 0
        cl = (i1_v[pl.ds(b, _L)] & 127)[0] + lane * 0
        dst = b * _EMBED
        for k in range(_EMBED // _L):
            val = plsc.load_gather(piece_v, [slot, lane + k * _L, cl])
            rows_v[pl.ds(dst + k * _L, _L)] = val

    def step(b, carry):
        @pl.when(b >= _RING)
        def _drain():
            wait_slot(b - _RING)
            extract(b - _RING)

        @pl.when(b < _BPW)
        def _fire():
            fire(b)

        return carry

    lax.fori_loop(0, _BPW + _RING, step, 0)

    pltpu.sync_copy(rows_v, out_hbm.at[pl.ds(base * _EMBED, _BPW * _EMBED)])


@jax.jit
def kernel(idx0, idx1, lookup):
    table_t = jnp.transpose(lookup, (0, 2, 1))
    run = pl.kernel(
        _gather_body,
        out_type=jax.ShapeDtypeStruct((_BATCH * _EMBED,), jnp.float32),
        mesh=plsc.VectorSubcoreMesh(core_axis_name="c", subcore_axis_name="s"),
        compiler_params=pltpu.CompilerParams(
            use_tc_tiling_on_sc=True, needs_layout_passes=False),
        scratch_types=[
            pltpu.VMEM((_BPW + _L,), jnp.int32),
            pltpu.VMEM((_BPW + _L,), jnp.int32),
            pltpu.VMEM((_RING, _EMBED, 128), jnp.float32),
            pltpu.VMEM((_BPW * _EMBED,), jnp.float32),
            pltpu.SemaphoreType.DMA,
        ],
    )
    out_flat = run(idx0, idx1, table_t)
    return out_flat.reshape(_BATCH, _EMBED)


# speedup vs baseline: 2.8620x; 1.0522x over previous
import jax
import jax.numpy as jnp
from jax import lax
from jax.experimental import pallas as pl
from jax.experimental.pallas import tpu as pltpu
from jax.experimental.pallas import tpu_sc as plsc

_VOCAB = 1000
_EMBED = 64
_BATCH = 16384

_NC = 2
_NS = 16
_NW = _NC * _NS
_BPW = _BATCH // _NW
_RING = 8
_L = 16


def _gather_body(idx0_hbm, idx1_hbm, table_hbm, out_hbm,
                 i0_v, i1_v, piece_v, rows_v, sem):
    wid = lax.axis_index("s") * _NC + lax.axis_index("c")
    base = wid * _BPW
    pltpu.sync_copy(idx0_hbm.at[pl.ds(base, _BPW)], i0_v.at[pl.ds(0, _BPW)])
    pltpu.sync_copy(idx1_hbm.at[pl.ds(base, _BPW)], i1_v.at[pl.ds(0, _BPW)])

    lane = lax.iota(jnp.int32, _L)

    def fire(b):
        i0 = i0_v[pl.ds(b, _L)][0]
        i1 = i1_v[pl.ds(b, _L)][0]
        ct = pl.multiple_of(lax.shift_right_logical(i1, 7) * 128, 128)
        pltpu.make_async_copy(
            table_hbm.at[i0, :, pl.ds(ct, 128)],
            piece_v.at[b & (_RING - 1)], sem,
        ).start()

    def wait_slot(b):
        pltpu.make_async_copy(
            table_hbm.at[0, :, pl.ds(pl.multiple_of(b * 0, 128), 128)],
            piece_v.at[b & (_RING - 1)], sem,
        ).wait()

    def extract(b):
        slot = (b & (_RING - 1)) + lane * 0
        cl = (i1_v[pl.ds(b, _L)] & 127)[0] + lane * 0
        dst = b * _EMBED
        for k in range(_EMBED // _L):
            val = plsc.load_gather(piece_v, [slot, lane + k * _L, cl])
            rows_v[pl.ds(dst + k * _L, _L)] = val

    def step(b, carry):
        @pl.when(b >= _RING)
        def _drain():
            wait_slot(b - _RING)
            extract(b - _RING)

        @pl.when(b < _BPW)
        def _fire():
            fire(b)

        return carry

    lax.fori_loop(0, _BPW + _RING, step, 0)

    pltpu.sync_copy(rows_v, out_hbm.at[pl.ds(base * _EMBED, _BPW * _EMBED)])


@jax.jit
def kernel(idx0, idx1, lookup):
    table_t = jnp.transpose(lookup, (0, 2, 1))
    run = pl.kernel(
        _gather_body,
        out_type=jax.ShapeDtypeStruct((_BATCH * _EMBED,), jnp.float32),
        mesh=plsc.VectorSubcoreMesh(core_axis_name="c", subcore_axis_name="s"),
        compiler_params=pltpu.CompilerParams(
            use_tc_tiling_on_sc=True, needs_layout_passes=False),
        scratch_types=[
            pltpu.VMEM((_BPW + _L,), jnp.int32),
            pltpu.VMEM((_BPW + _L,), jnp.int32),
            pltpu.VMEM((_RING, _EMBED, 128), jnp.float32),
            pltpu.VMEM((_BPW * _EMBED,), jnp.float32),
            pltpu.SemaphoreType.DMA,
        ],
    )
    out_flat = run(idx0, idx1, table_t)
    return out_flat.reshape(_BATCH, _EMBED)
